# baseline (speedup 1.0000x reference)
import jax
import jax.numpy as jnp
from jax import lax
from jax.experimental import pallas as pl
from jax.experimental.pallas import tpu as pltpu
from jax.experimental.pallas import tpu_sc as plsc

N, D, H, E, DOUT = 4096, 1024, 512, 8, 1024
EPS = 1e-5
NEG = -1e30
BLK = 128
NB = N // BLK
PAD = N * 2 + E * BLK
NBLK = PAD // BLK
CH = 128
_INV_SQRT2 = 0.7071067811865476

_NC, _NS = 2, 16
_NW = _NC * _NS
_TPW = N // _NW
_SCCH = 32
_NCHUNK = _TPW // _SCCH


def _gelu(x):
    return 0.5 * x * (1.0 + jax.lax.erf(x * _INV_SQRT2))


def _gate_body(x_ref, wg1_ref, bg1_ref, wg2_ref, bg2_ref,
               w0_ref, w1_ref, p0_ref, p1_ref, bexp_ref):
    x = x_ref[...]
    h = jnp.dot(x, wg1_ref[...], preferred_element_type=jnp.float32) + bg1_ref[...]
    h = _gelu(h)
    s = jnp.dot(h, wg2_ref[...], preferred_element_type=jnp.float32) + bg2_ref[...]
    lane = jax.lax.broadcasted_iota(jnp.int32, s.shape, 1)
    s0 = jnp.max(s, axis=1, keepdims=True)
    i0 = jnp.min(jnp.where(s == s0, lane, E), axis=1, keepdims=True)
    sm = jnp.where(lane == i0, NEG, s)
    s1 = jnp.max(sm, axis=1, keepdims=True)
    i1 = jnp.min(jnp.where(sm == s1, lane, E), axis=1, keepdims=True)
    e1 = jnp.exp(s1 - s0)
    w0 = 1.0 / (1.0 + e1)
    w0_ref[...] = w0
    w1_ref[...] = e1 * w0

    o0 = (lane == i0).astype(jnp.float32)
    o1 = (lane == i1).astype(jnp.float32)
    m = o0 + o1
    ri = jax.lax.broadcasted_iota(jnp.int32, (CH, CH), 0)
    ci = jax.lax.broadcasted_iota(jnp.int32, (CH, CH), 1)
    tri = (ri > ci).astype(jnp.float32)
    carry = jnp.zeros((1, E), jnp.float32)
    rank0 = []
    rank1 = []
    for c in range(N // CH):
        mc = m[c * CH:(c + 1) * CH, :]
        cc = jnp.dot(tri, mc, preferred_element_type=jnp.float32) + carry
        rank0.append(jnp.sum(cc * o0[c * CH:(c + 1) * CH, :], axis=1,
                             keepdims=True))
        rank1.append(jnp.sum(cc * o1[c * CH:(c + 1) * CH, :], axis=1,
                             keepdims=True))
        carry = carry + jnp.sum(mc, axis=0, keepdims=True)
    rank0 = jnp.concatenate(rank0, axis=0)
    rank1 = jnp.concatenate(rank1, axis=0)

    counts = carry
    pc = float(BLK) * jnp.floor((counts + float(BLK - 1)) / float(BLK))
    r8 = jax.lax.broadcasted_iota(jnp.int32, (E, E), 0)
    c8 = jax.lax.broadcasted_iota(jnp.int32, (E, E), 1)
    incl = (r8 <= c8).astype(jnp.float32)
    cumend = jnp.dot(pc, incl, preferred_element_type=jnp.float32)
    off = cumend - pc
    p0_ref[...] = (jnp.sum(o0 * off, axis=1, keepdims=True)
                   + rank0).astype(jnp.int32)
    p1_ref[...] = (jnp.sum(o1 * off, axis=1, keepdims=True)
                   + rank1).astype(jnp.int32)

    bi = (jax.lax.broadcasted_iota(jnp.int32, (NBLK, E), 0)
          .astype(jnp.float32) * float(BLK))
    cmp = (jnp.zeros((NBLK, E), jnp.float32) + cumend) <= bi
    bexp = jnp.sum(cmp.astype(jnp.int32), axis=1, keepdims=True)
    bexp_ref[...] = jnp.minimum(bexp, E - 1)


def _ragged_body(s_ref, xs_ref, we_ref, be_ref, g_ref, b_ref, o_ref):
    e = s_ref[pl.program_id(0)]
    y = jnp.dot(xs_ref[...], we_ref[e],
                preferred_element_type=jnp.float32) + be_ref[e]
    mu = jnp.mean(y, axis=1, keepdims=True)
    yc = y - mu
    var = jnp.mean(yc * yc, axis=1, keepdims=True)
    y = yc * jax.lax.rsqrt(var + EPS) * g_ref[e] + b_ref[e]
    o_ref[...] = _gelu(y)


def _combine_body(w0_ref, w1_ref, g0_ref, g1_ref, o_ref):
    o_ref[...] = w0_ref[...] * g0_ref[...] + w1_ref[...] * g1_ref[...]


def _sc_scatter(x_hbm, p0_hbm, p1_hbm, xs_hbm, i0_v, i1_v, row_a, row_b,
                sem_ld, sem_st):
    wid = lax.axis_index("s") * _NC + lax.axis_index("c")
    base = wid * _TPW
    r0 = wid * _NCHUNK
    pltpu.sync_copy(p0_hbm.at[pl.ds(r0, _NCHUNK)], i0_v)
    pltpu.sync_copy(p1_hbm.at[pl.ds(r0, _NCHUNK)], i1_v)
    pltpu.sync_copy(x_hbm.at[pl.ds(base, _SCCH)], row_a)
    ld = None
    for j in range(_NCHUNK):
        row = row_a if j % 2 == 0 else row_b
        if j < _NCHUNK - 1:
            nxt = row_b if j % 2 == 0 else row_a
            ld = pltpu.async_copy(
                x_hbm.at[pl.ds(base + (j + 1) * _SCCH, _SCCH)], nxt, sem_ld)
        s0 = pltpu.async_copy(row, xs_hbm.at[i0_v.at[j]], sem_st)
        s1 = pltpu.async_copy(row, xs_hbm.at[i1_v.at[j]], sem_st)
        s0.wait()
        s1.wait()
        if j < _NCHUNK - 1:
            ld.wait()


def _sc_gather(ys_hbm, p0_hbm, p1_hbm, g0_hbm, g1_hbm, i0_v, i1_v,
               row_a, row_b, sem_g):
    wid = lax.axis_index("s") * _NC + lax.axis_index("c")
    base = wid * _TPW
    r0 = wid * _NCHUNK
    pltpu.sync_copy(p0_hbm.at[pl.ds(r0, _NCHUNK)], i0_v)
    pltpu.sync_copy(p1_hbm.at[pl.ds(r0, _NCHUNK)], i1_v)
    for iv, g_hbm in ((i0_v, g0_hbm), (i1_v, g1_hbm)):
        cp = pltpu.async_copy(ys_hbm.at[iv.at[0]], row_a, sem_g)
        for j in range(_NCHUNK):
            cur = row_a if j % 2 == 0 else row_b
            cp.wait()
            if j < _NCHUNK - 1:
                nxt = row_b if j % 2 == 0 else row_a
                cp = pltpu.async_copy(ys_hbm.at[iv.at[j + 1]], nxt, sem_g)
            pltpu.sync_copy(cur, g_hbm.at[pl.ds(base + j * _SCCH, _SCCH)])


def kernel(x, Wg1, bg1, Wg2, bg2, We, be, gamma, beta):
    sc_mesh = plsc.VectorSubcoreMesh(core_axis_name="c", subcore_axis_name="s")
    w0, w1, p0, p1, bexp = pl.pallas_call(
        _gate_body,
        out_shape=[
            jax.ShapeDtypeStruct((N, 1), jnp.float32),
            jax.ShapeDtypeStruct((N, 1), jnp.float32),
            jax.ShapeDtypeStruct((N, 1), jnp.int32),
            jax.ShapeDtypeStruct((N, 1), jnp.int32),
            jax.ShapeDtypeStruct((NBLK, 1), jnp.int32),
        ],
    )(x, Wg1, bg1.reshape(1, H), Wg2, bg2.reshape(1, E))

    p0f = p0.reshape(N)
    p1f = p1.reshape(N)
    p0r = p0.reshape(N // _SCCH, _SCCH)
    p1r = p1.reshape(N // _SCCH, _SCCH)

    xs = pl.kernel(
        _sc_scatter,
        out_type=jax.ShapeDtypeStruct((PAD, D), jnp.float32),
        mesh=sc_mesh,
        scratch_types=[
            pltpu.VMEM((_NCHUNK, _SCCH), jnp.int32),
            pltpu.VMEM((_NCHUNK, _SCCH), jnp.int32),
            pltpu.VMEM((_SCCH, D), jnp.float32),
            pltpu.VMEM((_SCCH, D), jnp.float32),
            pltpu.SemaphoreType.DMA,
            pltpu.SemaphoreType.DMA,
        ],
    )(x, p0r, p1r)

    ys = pl.pallas_call(
        _ragged_body,
        grid_spec=pltpu.PrefetchScalarGridSpec(
            num_scalar_prefetch=1,
            grid=(NBLK,),
            in_specs=[
                pl.BlockSpec((BLK, D), lambda b, s: (b, 0)),
                pl.BlockSpec((E, D, DOUT), lambda b, s: (0, 0, 0)),
                pl.BlockSpec((E, 1, DOUT), lambda b, s: (0, 0, 0)),
                pl.BlockSpec((E, 1, DOUT), lambda b, s: (0, 0, 0)),
                pl.BlockSpec((E, 1, DOUT), lambda b, s: (0, 0, 0)),
            ],
            out_specs=pl.BlockSpec((BLK, DOUT), lambda b, s: (b, 0)),
        ),
        out_shape=jax.ShapeDtypeStruct((PAD, DOUT), jnp.float32),
        compiler_params=pltpu.CompilerParams(
            dimension_semantics=("arbitrary",)),
    )(bexp.reshape(NBLK), xs, We, be.reshape(E, 1, DOUT),
      gamma.reshape(E, 1, DOUT), beta.reshape(E, 1, DOUT))

    g0, g1 = pl.kernel(
        _sc_gather,
        out_type=[
            jax.ShapeDtypeStruct((N, DOUT), jnp.float32),
            jax.ShapeDtypeStruct((N, DOUT), jnp.float32),
        ],
        mesh=sc_mesh,
        scratch_types=[
            pltpu.VMEM((_NCHUNK, _SCCH), jnp.int32),
            pltpu.VMEM((_NCHUNK, _SCCH), jnp.int32),
            pltpu.VMEM((_SCCH, DOUT), jnp.float32),
            pltpu.VMEM((_SCCH, DOUT), jnp.float32),
            pltpu.SemaphoreType.DMA,
        ],
    )(ys, p0r, p1r)

    cblk = 512
    out = pl.pallas_call(
        _combine_body,
        grid=(N // cblk,),
        in_specs=[
            pl.BlockSpec((cblk, 1), lambda i: (i, 0)),
            pl.BlockSpec((cblk, 1), lambda i: (i, 0)),
            pl.BlockSpec((cblk, DOUT), lambda i: (i, 0)),
            pl.BlockSpec((cblk, DOUT), lambda i: (i, 0)),
        ],
        out_specs=pl.BlockSpec((cblk, DOUT), lambda i: (i, 0)),
        out_shape=jax.ShapeDtypeStruct((N, DOUT), jnp.float32),
    )(w0, w1, g0, g1)
    return out


# speedup vs baseline: 1.0988x; 1.0988x over previous
import jax
import jax.numpy as jnp
from jax import lax
from jax.experimental import pallas as pl
from jax.experimental.pallas import tpu as pltpu
from jax.experimental.pallas import tpu_sc as plsc

N, D, H, E, DOUT = 4096, 1024, 512, 8, 1024
EPS = 1e-5
NEG = -1e30
BLK = 256
NB = N // BLK
PAD = N * 2 + E * BLK
NBLK = PAD // BLK
CH = 128
_INV_SQRT2 = 0.7071067811865476

_NC, _NS = 2, 16
_NW = _NC * _NS
_TPW = N // _NW
_SCCH = 32
_NCHUNK = _TPW // _SCCH


def _gelu(x):
    return 0.5 * x * (1.0 + jax.lax.erf(x * _INV_SQRT2))


def _gate_body(x_ref, wg1_ref, bg1_ref, wg2_ref, bg2_ref,
               w0_ref, w1_ref, p0_ref, p1_ref, bexp_ref):
    x = x_ref[...]
    h = jnp.dot(x, wg1_ref[...], preferred_element_type=jnp.float32) + bg1_ref[...]
    h = _gelu(h)
    s = jnp.dot(h, wg2_ref[...], preferred_element_type=jnp.float32) + bg2_ref[...]
    lane = jax.lax.broadcasted_iota(jnp.int32, s.shape, 1)
    s0 = jnp.max(s, axis=1, keepdims=True)
    i0 = jnp.min(jnp.where(s == s0, lane, E), axis=1, keepdims=True)
    sm = jnp.where(lane == i0, NEG, s)
    s1 = jnp.max(sm, axis=1, keepdims=True)
    i1 = jnp.min(jnp.where(sm == s1, lane, E), axis=1, keepdims=True)
    e1 = jnp.exp(s1 - s0)
    w0 = 1.0 / (1.0 + e1)
    w0_ref[...] = w0
    w1_ref[...] = e1 * w0

    o0 = (lane == i0).astype(jnp.float32)
    o1 = (lane == i1).astype(jnp.float32)
    m = o0 + o1
    ri = jax.lax.broadcasted_iota(jnp.int32, (CH, CH), 0)
    ci = jax.lax.broadcasted_iota(jnp.int32, (CH, CH), 1)
    tri = (ri > ci).astype(jnp.float32)
    carry = jnp.zeros((1, E), jnp.float32)
    rank0 = []
    rank1 = []
    for c in range(N // CH):
        mc = m[c * CH:(c + 1) * CH, :]
        cc = jnp.dot(tri, mc, preferred_element_type=jnp.float32) + carry
        rank0.append(jnp.sum(cc * o0[c * CH:(c + 1) * CH, :], axis=1,
                             keepdims=True))
        rank1.append(jnp.sum(cc * o1[c * CH:(c + 1) * CH, :], axis=1,
                             keepdims=True))
        carry = carry + jnp.sum(mc, axis=0, keepdims=True)
    rank0 = jnp.concatenate(rank0, axis=0)
    rank1 = jnp.concatenate(rank1, axis=0)

    counts = carry
    pc = float(BLK) * jnp.floor((counts + float(BLK - 1)) / float(BLK))
    r8 = jax.lax.broadcasted_iota(jnp.int32, (E, E), 0)
    c8 = jax.lax.broadcasted_iota(jnp.int32, (E, E), 1)
    incl = (r8 <= c8).astype(jnp.float32)
    cumend = jnp.dot(pc, incl, preferred_element_type=jnp.float32)
    off = cumend - pc
    p0_ref[...] = (jnp.sum(o0 * off, axis=1, keepdims=True)
                   + rank0).astype(jnp.int32)
    p1_ref[...] = (jnp.sum(o1 * off, axis=1, keepdims=True)
                   + rank1).astype(jnp.int32)

    bi = (jax.lax.broadcasted_iota(jnp.int32, (NBLK, E), 0)
          .astype(jnp.float32) * float(BLK))
    cmp = (jnp.zeros((NBLK, E), jnp.float32) + cumend) <= bi
    bexp = jnp.sum(cmp.astype(jnp.int32), axis=1, keepdims=True)
    bexp_ref[...] = jnp.minimum(bexp, E - 1)


def _ragged_body(s_ref, xs_ref, we_ref, be_ref, g_ref, b_ref, o_ref):
    e = s_ref[pl.program_id(0)]
    y = jnp.dot(xs_ref[...], we_ref[e],
                preferred_element_type=jnp.float32) + be_ref[e]
    mu = jnp.mean(y, axis=1, keepdims=True)
    yc = y - mu
    var = jnp.mean(yc * yc, axis=1, keepdims=True)
    y = yc * jax.lax.rsqrt(var + EPS) * g_ref[e] + b_ref[e]
    o_ref[...] = _gelu(y)


def _combine_body(w0_ref, w1_ref, g0_ref, g1_ref, o_ref):
    o_ref[...] = w0_ref[...] * g0_ref[...] + w1_ref[...] * g1_ref[...]


def _sc_scatter(x_hbm, p0_hbm, p1_hbm, xs_hbm, i0_v, i1_v, row_a, row_b,
                sem_ld, sem_st):
    wid = lax.axis_index("s") * _NC + lax.axis_index("c")
    base = wid * _TPW
    r0 = wid * _NCHUNK
    pltpu.sync_copy(p0_hbm.at[pl.ds(r0, _NCHUNK)], i0_v)
    pltpu.sync_copy(p1_hbm.at[pl.ds(r0, _NCHUNK)], i1_v)
    pltpu.sync_copy(x_hbm.at[pl.ds(base, _SCCH)], row_a)
    ld = None
    for j in range(_NCHUNK):
        row = row_a if j % 2 == 0 else row_b
        if j < _NCHUNK - 1:
            nxt = row_b if j % 2 == 0 else row_a
            ld = pltpu.async_copy(
                x_hbm.at[pl.ds(base + (j + 1) * _SCCH, _SCCH)], nxt, sem_ld)
        s0 = pltpu.async_copy(row, xs_hbm.at[i0_v.at[j]], sem_st)
        s1 = pltpu.async_copy(row, xs_hbm.at[i1_v.at[j]], sem_st)
        s0.wait()
        s1.wait()
        if j < _NCHUNK - 1:
            ld.wait()


def _sc_gather(ys_hbm, p0_hbm, p1_hbm, g0_hbm, g1_hbm, i0_v, i1_v,
               row_a, row_b, sem_g):
    wid = lax.axis_index("s") * _NC + lax.axis_index("c")
    base = wid * _TPW
    r0 = wid * _NCHUNK
    pltpu.sync_copy(p0_hbm.at[pl.ds(r0, _NCHUNK)], i0_v)
    pltpu.sync_copy(p1_hbm.at[pl.ds(r0, _NCHUNK)], i1_v)
    for iv, g_hbm in ((i0_v, g0_hbm), (i1_v, g1_hbm)):
        cp = pltpu.async_copy(ys_hbm.at[iv.at[0]], row_a, sem_g)
        for j in range(_NCHUNK):
            cur = row_a if j % 2 == 0 else row_b
            cp.wait()
            if j < _NCHUNK - 1:
                nxt = row_b if j % 2 == 0 else row_a
                cp = pltpu.async_copy(ys_hbm.at[iv.at[j + 1]], nxt, sem_g)
            pltpu.sync_copy(cur, g_hbm.at[pl.ds(base + j * _SCCH, _SCCH)])


def kernel(x, Wg1, bg1, Wg2, bg2, We, be, gamma, beta):
    sc_mesh = plsc.VectorSubcoreMesh(core_axis_name="c", subcore_axis_name="s")
    w0, w1, p0, p1, bexp = pl.pallas_call(
        _gate_body,
        out_shape=[
            jax.ShapeDtypeStruct((N, 1), jnp.float32),
            jax.ShapeDtypeStruct((N, 1), jnp.float32),
            jax.ShapeDtypeStruct((N, 1), jnp.int32),
            jax.ShapeDtypeStruct((N, 1), jnp.int32),
            jax.ShapeDtypeStruct((NBLK, 1), jnp.int32),
        ],
    )(x, Wg1, bg1.reshape(1, H), Wg2, bg2.reshape(1, E))

    p0f = p0.reshape(N)
    p1f = p1.reshape(N)
    p0r = p0.reshape(N // _SCCH, _SCCH)
    p1r = p1.reshape(N // _SCCH, _SCCH)

    xs = pl.kernel(
        _sc_scatter,
        out_type=jax.ShapeDtypeStruct((PAD, D), jnp.float32),
        mesh=sc_mesh,
        scratch_types=[
            pltpu.VMEM((_NCHUNK, _SCCH), jnp.int32),
            pltpu.VMEM((_NCHUNK, _SCCH), jnp.int32),
            pltpu.VMEM((_SCCH, D), jnp.float32),
            pltpu.VMEM((_SCCH, D), jnp.float32),
            pltpu.SemaphoreType.DMA,
            pltpu.SemaphoreType.DMA,
        ],
    )(x, p0r, p1r)

    ys = pl.pallas_call(
        _ragged_body,
        grid_spec=pltpu.PrefetchScalarGridSpec(
            num_scalar_prefetch=1,
            grid=(NBLK,),
            in_specs=[
                pl.BlockSpec((BLK, D), lambda b, s: (b, 0)),
                pl.BlockSpec((E, D, DOUT), lambda b, s: (0, 0, 0)),
                pl.BlockSpec((E, 1, DOUT), lambda b, s: (0, 0, 0)),
                pl.BlockSpec((E, 1, DOUT), lambda b, s: (0, 0, 0)),
                pl.BlockSpec((E, 1, DOUT), lambda b, s: (0, 0, 0)),
            ],
            out_specs=pl.BlockSpec((BLK, DOUT), lambda b, s: (b, 0)),
        ),
        out_shape=jax.ShapeDtypeStruct((PAD, DOUT), jnp.float32),
        compiler_params=pltpu.CompilerParams(
            dimension_semantics=("arbitrary",)),
    )(bexp.reshape(NBLK), xs, We, be.reshape(E, 1, DOUT),
      gamma.reshape(E, 1, DOUT), beta.reshape(E, 1, DOUT))

    g0, g1 = pl.kernel(
        _sc_gather,
        out_type=[
            jax.ShapeDtypeStruct((N, DOUT), jnp.float32),
            jax.ShapeDtypeStruct((N, DOUT), jnp.float32),
        ],
        mesh=sc_mesh,
        scratch_types=[
            pltpu.VMEM((_NCHUNK, _SCCH), jnp.int32),
            pltpu.VMEM((_NCHUNK, _SCCH), jnp.int32),
            pltpu.VMEM((_SCCH, DOUT), jnp.float32),
            pltpu.VMEM((_SCCH, DOUT), jnp.float32),
            pltpu.SemaphoreType.DMA,
        ],
    )(ys, p0r, p1r)

    cblk = 512
    out = pl.pallas_call(
        _combine_body,
        grid=(N // cblk,),
        in_specs=[
            pl.BlockSpec((cblk, 1), lambda i: (i, 0)),
            pl.BlockSpec((cblk, 1), lambda i: (i, 0)),
            pl.BlockSpec((cblk, DOUT), lambda i: (i, 0)),
            pl.BlockSpec((cblk, DOUT), lambda i: (i, 0)),
        ],
        out_specs=pl.BlockSpec((cblk, DOUT), lambda i: (i, 0)),
        out_shape=jax.ShapeDtypeStruct((N, DOUT), jnp.float32),
    )(w0, w1, g0, g1)
    return out


# speedup vs baseline: 1.0994x; 1.0006x over previous
import jax
import jax.numpy as jnp
from jax import lax
from jax.experimental import pallas as pl
from jax.experimental.pallas import tpu as pltpu
from jax.experimental.pallas import tpu_sc as plsc

N, D, H, E, DOUT = 4096, 1024, 512, 8, 1024
EPS = 1e-5
NEG = -1e30
BLK = 256
NB = N // BLK
PAD = N * 2 + E * BLK
NBLK = PAD // BLK
CH = 128
_INV_SQRT2 = 0.7071067811865476

_NC, _NS = 2, 16
_NW = _NC * _NS
_TPW = N // _NW
_SCCH = 32
_NCHUNK = _TPW // _SCCH


def _gelu(x):
    return 0.5 * x * (1.0 + jax.lax.erf(x * _INV_SQRT2))


def _gate_body(x_ref, wg1_ref, bg1_ref, wg2_ref, bg2_ref,
               w0_ref, w1_ref, p0_ref, p1_ref, bexp_ref):
    x = x_ref[...]
    h = jnp.dot(x, wg1_ref[...], preferred_element_type=jnp.float32) + bg1_ref[...]
    h = _gelu(h)
    s = jnp.dot(h, wg2_ref[...], preferred_element_type=jnp.float32) + bg2_ref[...]
    lane = jax.lax.broadcasted_iota(jnp.int32, s.shape, 1)
    s0 = jnp.max(s, axis=1, keepdims=True)
    i0 = jnp.min(jnp.where(s == s0, lane, E), axis=1, keepdims=True)
    sm = jnp.where(lane == i0, NEG, s)
    s1 = jnp.max(sm, axis=1, keepdims=True)
    i1 = jnp.min(jnp.where(sm == s1, lane, E), axis=1, keepdims=True)
    e1 = jnp.exp(s1 - s0)
    w0 = 1.0 / (1.0 + e1)
    w0_ref[...] = w0
    w1_ref[...] = e1 * w0

    o0 = (lane == i0).astype(jnp.float32)
    o1 = (lane == i1).astype(jnp.float32)
    m = o0 + o1
    ri = jax.lax.broadcasted_iota(jnp.int32, (CH, CH), 0)
    ci = jax.lax.broadcasted_iota(jnp.int32, (CH, CH), 1)
    tri = (ri > ci).astype(jnp.float32)
    carry = jnp.zeros((1, E), jnp.float32)
    rank0 = []
    rank1 = []
    for c in range(N // CH):
        mc = m[c * CH:(c + 1) * CH, :]
        cc = jnp.dot(tri, mc, preferred_element_type=jnp.float32) + carry
        rank0.append(jnp.sum(cc * o0[c * CH:(c + 1) * CH, :], axis=1,
                             keepdims=True))
        rank1.append(jnp.sum(cc * o1[c * CH:(c + 1) * CH, :], axis=1,
                             keepdims=True))
        carry = carry + jnp.sum(mc, axis=0, keepdims=True)
    rank0 = jnp.concatenate(rank0, axis=0)
    rank1 = jnp.concatenate(rank1, axis=0)

    counts = carry
    pc = float(BLK) * jnp.floor((counts + float(BLK - 1)) / float(BLK))
    r8 = jax.lax.broadcasted_iota(jnp.int32, (E, E), 0)
    c8 = jax.lax.broadcasted_iota(jnp.int32, (E, E), 1)
    incl = (r8 <= c8).astype(jnp.float32)
    cumend = jnp.dot(pc, incl, preferred_element_type=jnp.float32)
    off = cumend - pc
    p0_ref[...] = (jnp.sum(o0 * off, axis=1, keepdims=True)
                   + rank0).astype(jnp.int32)
    p1_ref[...] = (jnp.sum(o1 * off, axis=1, keepdims=True)
                   + rank1).astype(jnp.int32)

    bi = (jax.lax.broadcasted_iota(jnp.int32, (NBLK, E), 0)
          .astype(jnp.float32) * float(BLK))
    cmp = (jnp.zeros((NBLK, E), jnp.float32) + cumend) <= bi
    bexp = jnp.sum(cmp.astype(jnp.int32), axis=1, keepdims=True)
    bexp_ref[...] = jnp.minimum(bexp, E - 1)


def _ragged_body(s_ref, xs_ref, we_ref, be_ref, g_ref, b_ref, o_ref):
    e = s_ref[pl.program_id(0)]
    y = jnp.dot(xs_ref[...], we_ref[e],
                preferred_element_type=jnp.float32) + be_ref[e]
    mu = jnp.mean(y, axis=1, keepdims=True)
    yc = y - mu
    var = jnp.mean(yc * yc, axis=1, keepdims=True)
    y = yc * jax.lax.rsqrt(var + EPS) * g_ref[e] + b_ref[e]
    o_ref[...] = _gelu(y)


def _combine_body(w0_ref, w1_ref, g0_ref, g1_ref, o_ref):
    o_ref[...] = w0_ref[...] * g0_ref[...] + w1_ref[...] * g1_ref[...]


def _sc_scatter(x_hbm, p0_hbm, p1_hbm, xs_hbm, i0_v, i1_v, row_a, row_b,
                sem_ld, sem_st):
    wid = lax.axis_index("s") * _NC + lax.axis_index("c")
    base = wid * _TPW
    r0 = wid * _NCHUNK
    pltpu.sync_copy(p0_hbm.at[pl.ds(r0, _NCHUNK)], i0_v)
    pltpu.sync_copy(p1_hbm.at[pl.ds(r0, _NCHUNK)], i1_v)
    pltpu.sync_copy(x_hbm.at[pl.ds(base, _SCCH)], row_a)
    ld = None
    for j in range(_NCHUNK):
        row = row_a if j % 2 == 0 else row_b
        if j < _NCHUNK - 1:
            nxt = row_b if j % 2 == 0 else row_a
            ld = pltpu.async_copy(
                x_hbm.at[pl.ds(base + (j + 1) * _SCCH, _SCCH)], nxt, sem_ld)
        s0 = pltpu.async_copy(row, xs_hbm.at[i0_v.at[j]], sem_st)
        s1 = pltpu.async_copy(row, xs_hbm.at[i1_v.at[j]], sem_st)
        s0.wait()
        s1.wait()
        if j < _NCHUNK - 1:
            ld.wait()


def _sc_gather(ys_hbm, p0_hbm, p1_hbm, g0_hbm, g1_hbm, i0_v, i1_v,
               row_a, row_b, sem_g):
    wid = lax.axis_index("s") * _NC + lax.axis_index("c")
    base = wid * _TPW
    r0 = wid * _NCHUNK
    pltpu.sync_copy(p0_hbm.at[pl.ds(r0, _NCHUNK)], i0_v)
    pltpu.sync_copy(p1_hbm.at[pl.ds(r0, _NCHUNK)], i1_v)
    for iv, g_hbm in ((i0_v, g0_hbm), (i1_v, g1_hbm)):
        cp = pltpu.async_copy(ys_hbm.at[iv.at[0]], row_a, sem_g)
        for j in range(_NCHUNK):
            cur = row_a if j % 2 == 0 else row_b
            cp.wait()
            if j < _NCHUNK - 1:
                nxt = row_b if j % 2 == 0 else row_a
                cp = pltpu.async_copy(ys_hbm.at[iv.at[j + 1]], nxt, sem_g)
            pltpu.sync_copy(cur, g_hbm.at[pl.ds(base + j * _SCCH, _SCCH)])


def kernel(x, Wg1, bg1, Wg2, bg2, We, be, gamma, beta):
    sc_mesh = plsc.VectorSubcoreMesh(core_axis_name="c", subcore_axis_name="s")
    w0, w1, p0, p1, bexp = pl.pallas_call(
        _gate_body,
        out_shape=[
            jax.ShapeDtypeStruct((N, 1), jnp.float32),
            jax.ShapeDtypeStruct((N, 1), jnp.float32),
            jax.ShapeDtypeStruct((N, 1), jnp.int32),
            jax.ShapeDtypeStruct((N, 1), jnp.int32),
            jax.ShapeDtypeStruct((NBLK, 1), jnp.int32),
        ],
    )(x, Wg1, bg1.reshape(1, H), Wg2, bg2.reshape(1, E))

    p0r = p0.reshape(N // _SCCH, _SCCH)
    p1r = p1.reshape(N // _SCCH, _SCCH)

    xs = pl.kernel(
        _sc_scatter,
        out_type=jax.ShapeDtypeStruct((PAD, D), jnp.float32),
        mesh=sc_mesh,
        scratch_types=[
            pltpu.VMEM((_NCHUNK, _SCCH), jnp.int32),
            pltpu.VMEM((_NCHUNK, _SCCH), jnp.int32),
            pltpu.VMEM((_SCCH, D), jnp.float32),
            pltpu.VMEM((_SCCH, D), jnp.float32),
            pltpu.SemaphoreType.DMA,
            pltpu.SemaphoreType.DMA,
        ],
    )(x, p0r, p1r)

    ys = pl.pallas_call(
        _ragged_body,
        grid_spec=pltpu.PrefetchScalarGridSpec(
            num_scalar_prefetch=1,
            grid=(NBLK,),
            in_specs=[
                pl.BlockSpec((BLK, D), lambda b, s: (b, 0)),
                pl.BlockSpec((E, D, DOUT), lambda b, s: (0, 0, 0)),
                pl.BlockSpec((E, 1, DOUT), lambda b, s: (0, 0, 0)),
                pl.BlockSpec((E, 1, DOUT), lambda b, s: (0, 0, 0)),
                pl.BlockSpec((E, 1, DOUT), lambda b, s: (0, 0, 0)),
            ],
            out_specs=pl.BlockSpec((BLK, DOUT), lambda b, s: (b, 0)),
        ),
        out_shape=jax.ShapeDtypeStruct((PAD, DOUT), jnp.float32),
        compiler_params=pltpu.CompilerParams(
            dimension_semantics=("arbitrary",)),
    )(bexp.reshape(NBLK), xs, We, be.reshape(E, 1, DOUT),
      gamma.reshape(E, 1, DOUT), beta.reshape(E, 1, DOUT))

    g0, g1 = pl.kernel(
        _sc_gather,
        out_type=[
            jax.ShapeDtypeStruct((N, DOUT), jnp.float32),
            jax.ShapeDtypeStruct((N, DOUT), jnp.float32),
        ],
        mesh=sc_mesh,
        scratch_types=[
            pltpu.VMEM((_NCHUNK, _SCCH), jnp.int32),
            pltpu.VMEM((_NCHUNK, _SCCH), jnp.int32),
            pltpu.VMEM((_SCCH, DOUT), jnp.float32),
            pltpu.VMEM((_SCCH, DOUT), jnp.float32),
            pltpu.SemaphoreType.DMA,
        ],
    )(ys, p0r, p1r)

    cblk = 512
    out = pl.pallas_call(
        _combine_body,
        grid=(N // cblk,),
        in_specs=[
            pl.BlockSpec((cblk, 1), lambda i: (i, 0)),
            pl.BlockSpec((cblk, 1), lambda i: (i, 0)),
            pl.BlockSpec((cblk, DOUT), lambda i: (i, 0)),
            pl.BlockSpec((cblk, DOUT), lambda i: (i, 0)),
        ],
        out_specs=pl.BlockSpec((cblk, DOUT), lambda i: (i, 0)),
        out_shape=jax.ShapeDtypeStruct((N, DOUT), jnp.float32),
    )(w0, w1, g0, g1)
    return out
